# baseline (speedup 1.0000x reference)
import functools

import jax
import jax.numpy as jnp
from jax import lax
from jax.experimental import pallas as pl
from jax.experimental.pallas import tpu as pltpu
from jax.experimental.pallas import tpu_sc as plsc

N = 10000
E = 320000
D = 128
G = 128

NPAD = 10240
EPAD = 327680
ER = EPAD // 128
TILES = 16
ER_T = ER // TILES
NR_T = NPAD // TILES

_MESH = plsc.VectorSubcoreMesh(core_axis_name="c", subcore_axis_name="s")


@functools.partial(
    pl.kernel,
    out_type=jax.ShapeDtypeStruct((4 * NPAD,), jnp.float32),
    mesh=_MESH,
    scratch_types=[
        pltpu.VMEM((ER_T, 128), jnp.int32),
        pltpu.VMEM((ER_T, 128), jnp.int32),
        pltpu.VMEM((ER_T, 128), jnp.float32),
        pltpu.VMEM((NR_T,), jnp.float32),
        pltpu.VMEM_SHARED((NPAD,), jnp.float32),
        pltpu.VMEM_SHARED((NPAD,), jnp.float32),
    ],
)
def _deg_kernel(src_hbm, dst_hbm, vals_hbm, deg_out,
                src_v, dst_v, vals_v, buf_v, hist_s, hist_d):
    c = lax.axis_index("c")
    s = lax.axis_index("s")

    def zero16(i, _):
        buf_v[pl.ds(i * 16, 16)] = jnp.zeros((16,), jnp.float32)
        return _

    lax.fori_loop(0, NR_T // 16, zero16, None)
    pltpu.sync_copy(buf_v, hist_s.at[pl.ds(s * NR_T, NR_T)])
    pltpu.sync_copy(buf_v, hist_d.at[pl.ds(s * NR_T, NR_T)])
    pltpu.sync_copy(src_hbm.at[pl.ds(s * ER_T, ER_T)], src_v)
    pltpu.sync_copy(dst_hbm.at[pl.ds(s * ER_T, ER_T)], dst_v)
    pltpu.sync_copy(vals_hbm.at[pl.ds((c * TILES + s) * ER_T, ER_T)], vals_v)
    plsc.subcore_barrier()

    def scat(j, _):
        pltpu.sync_copy(vals_v.at[j], hist_s.at[src_v.at[j]], add=True)
        pltpu.sync_copy(vals_v.at[j], hist_d.at[dst_v.at[j]], add=True)
        return _

    lax.fori_loop(0, ER_T, scat, None)
    plsc.subcore_barrier()
    pltpu.sync_copy(hist_s.at[pl.ds(s * NR_T, NR_T)], buf_v)
    pltpu.sync_copy(buf_v, deg_out.at[pl.ds((2 * c) * NPAD + s * NR_T, NR_T)])
    pltpu.sync_copy(hist_d.at[pl.ds(s * NR_T, NR_T)], buf_v)
    pltpu.sync_copy(buf_v, deg_out.at[pl.ds((2 * c + 1) * NPAD + s * NR_T, NR_T)])


_CH = 80
_NBUF = 4
_CH_T = EPAD // TILES // _CH
_GRP = _CH_T // _NBUF
_IDXROWS = 16


@functools.partial(
    pl.kernel,
    out_type=jax.ShapeDtypeStruct((2 * NPAD, D), jnp.float32),
    mesh=_MESH,
    scratch_types=[
        pltpu.VMEM((_IDXROWS, _CH), jnp.int32),
        pltpu.VMEM((_IDXROWS, _CH), jnp.int32),
        pltpu.VMEM((_CH, D), jnp.float32),
        pltpu.VMEM((_CH, D), jnp.float32),
        pltpu.VMEM((_CH, D), jnp.float32),
        pltpu.VMEM((_CH, D), jnp.float32),
        pltpu.VMEM_SHARED((NPAD, D), jnp.float32),
        pltpu.SemaphoreType.DMA,
        pltpu.SemaphoreType.DMA,
        pltpu.SemaphoreType.DMA,
        pltpu.SemaphoreType.DMA,
    ],
)
def _agg_kernel(xs1_hbm, xs2_hbm, srcsel_hbm, dst_hbm, zblk_hbm, agg_out,
                src_v, dst_v, rows0, rows1, rows2, rows3, acc,
                sem0, sem1, sem2, sem3):
    c = lax.axis_index("c")
    s = lax.axis_index("s")
    rows = (rows0, rows1, rows2, rows3)
    sems = (sem0, sem1, sem2, sem3)

    pltpu.sync_copy(zblk_hbm, rows0)

    def zero_acc(k, _):
        pltpu.sync_copy(rows0, acc.at[pl.ds(s * NR_T + k * _CH, _CH)])
        return _

    lax.fori_loop(0, NR_T // _CH, zero_acc, None)
    plsc.subcore_barrier()

    def run(table_hbm):
        def group(gk, _):
            @pl.when(gk % (_IDXROWS // _NBUF) == 0)
            def _():
                t = gk // (_IDXROWS // _NBUF)
                pltpu.sync_copy(
                    srcsel_hbm.at[
                        pl.ds((c * TILES + s) * _CH_T + t * _IDXROWS, _IDXROWS)],
                    src_v)
                pltpu.sync_copy(
                    dst_hbm.at[pl.ds(s * _CH_T + t * _IDXROWS, _IDXROWS)],
                    dst_v)

            base = (gk % (_IDXROWS // _NBUF)) * _NBUF
            gathers = []
            for b in range(_NBUF):
                @pl.when(gk > 0)
                def _(b=b):
                    pltpu.make_async_copy(zblk_hbm, rows[b], sems[b]).wait()

                gathers.append(pltpu.async_copy(
                    table_hbm.at[src_v.at[base + b]], rows[b], sems[b]))
            for b in range(_NBUF):
                gathers[b].wait()
                pltpu.async_copy(
                    rows[b], acc.at[dst_v.at[base + b]], sems[b], add=True)
            return _

        lax.fori_loop(0, _GRP, group, None)
        for b in range(_NBUF):
            pltpu.make_async_copy(zblk_hbm, rows[b], sems[b]).wait()

    @pl.when(c == 0)
    def _():
        run(xs1_hbm)

    @pl.when(c == 1)
    def _():
        run(xs2_hbm)

    plsc.subcore_barrier()

    def copy_out(k, _):
        base = s * NR_T + k * _CH
        pltpu.sync_copy(acc.at[pl.ds(base, _CH)], rows0)
        pltpu.sync_copy(rows0, agg_out.at[pl.ds(c * NPAD + base, _CH)])
        return _

    lax.fori_loop(0, NR_T // _CH, copy_out, None)


_PREP_STEPS = 10
_RB = NPAD // _PREP_STEPS
_EB = ER // _PREP_STEPS


def _prep_body(xpad_ref, degt_ref, fm_ref, batch_ref, src_ref, ew_ref,
               xs1_ref, xs2_ref, src2_ref, x1_ref, m2_ref, m1_ref):
    i = pl.program_id(0)
    xb = xpad_ref[...]
    inv1 = lax.rsqrt(jnp.maximum(degt_ref[:, 0:1], 1.0))
    inv2 = lax.rsqrt(jnp.maximum(degt_ref[:, 2:3], 1.0))
    xs1_ref[...] = xb * inv1
    xs2_ref[...] = xb * inv2
    x1_ref[...] = xb * fm_ref[...]
    zrow = N + lax.broadcasted_iota(jnp.int32, (_EB, 128), 1)
    src2_ref[...] = jnp.where(ew_ref[...] > 0.0, src_ref[...], zrow)
    gids = lax.broadcasted_iota(jnp.int32, (G, _RB), 0)
    oh = (gids == batch_ref[...]).astype(jnp.float32)

    @pl.when(i == 0)
    def _():
        m2_ref[...] = jnp.zeros((G, D), jnp.float32)

    m2_ref[...] += jnp.dot(oh, xb, preferred_element_type=jnp.float32)

    @pl.when(i == _PREP_STEPS - 1)
    def _():
        m1_ref[...] = m2_ref[...] * fm_ref[...]


def _prep_call(xpad, degt, fm, batchpad, src_pad, ew_pad):
    return pl.pallas_call(
        _prep_body,
        grid=(_PREP_STEPS,),
        in_specs=[
            pl.BlockSpec((_RB, D), lambda i: (i, 0)),
            pl.BlockSpec((_RB, 4), lambda i: (i, 0)),
            pl.BlockSpec((1, D), lambda i: (0, 0)),
            pl.BlockSpec((1, _RB), lambda i: (0, i)),
            pl.BlockSpec((_EB, 128), lambda i: (i, 0)),
            pl.BlockSpec((_EB, 128), lambda i: (i, 0)),
        ],
        out_specs=[
            pl.BlockSpec((_RB, D), lambda i: (i, 0)),
            pl.BlockSpec((_RB, D), lambda i: (i, 0)),
            pl.BlockSpec((_EB, 128), lambda i: (i, 0)),
            pl.BlockSpec((_RB, D), lambda i: (i, 0)),
            pl.BlockSpec((G, D), lambda i: (0, 0)),
            pl.BlockSpec((G, D), lambda i: (0, 0)),
        ],
        out_shape=[
            jax.ShapeDtypeStruct((NPAD, D), jnp.float32),
            jax.ShapeDtypeStruct((NPAD, D), jnp.float32),
            jax.ShapeDtypeStruct((ER, 128), jnp.int32),
            jax.ShapeDtypeStruct((NPAD, D), jnp.float32),
            jax.ShapeDtypeStruct((G, D), jnp.float32),
            jax.ShapeDtypeStruct((G, D), jnp.float32),
        ],
    )(xpad, degt, fm, batchpad, src_pad, ew_pad)


def _post_body(agg1_ref, agg2_ref, degt_ref, fm_ref, w_ref, b_ref, batch_ref,
               z_ref, z1_ref, z2_ref, g_ref, g1_ref, g2_ref):
    i = pl.program_id(0)
    invd1 = lax.rsqrt(jnp.maximum(degt_ref[:, 1:2], 1.0))
    invd2 = lax.rsqrt(jnp.maximum(degt_ref[:, 3:4], 1.0))
    a1 = agg1_ref[...] * invd1
    a2 = agg2_ref[...] * invd2
    w = w_ref[...]
    b = b_ref[...]
    z = jnp.maximum(jnp.dot(a1, w, preferred_element_type=jnp.float32) + b, 0.0)
    z1 = jnp.maximum(
        jnp.dot(a1 * fm_ref[...], w, preferred_element_type=jnp.float32) + b, 0.0)
    z2 = jnp.maximum(jnp.dot(a2, w, preferred_element_type=jnp.float32) + b, 0.0)
    z_ref[...] = z
    z1_ref[...] = z1
    z2_ref[...] = z2
    gids = lax.broadcasted_iota(jnp.int32, (G, _RB), 0)
    oh = (gids == batch_ref[...]).astype(jnp.float32)

    @pl.when(i == 0)
    def _():
        g_ref[...] = jnp.zeros((G, D), jnp.float32)
        g1_ref[...] = jnp.zeros((G, D), jnp.float32)
        g2_ref[...] = jnp.zeros((G, D), jnp.float32)

    g_ref[...] += jnp.dot(oh, z, preferred_element_type=jnp.float32)
    g1_ref[...] += jnp.dot(oh, z1, preferred_element_type=jnp.float32)
    g2_ref[...] += jnp.dot(oh, z2, preferred_element_type=jnp.float32)


def _post_call(aggs, degt, fm, w, b2, batchpad):
    return pl.pallas_call(
        _post_body,
        grid=(_PREP_STEPS,),
        in_specs=[
            pl.BlockSpec((_RB, D), lambda i: (i, 0)),
            pl.BlockSpec((_RB, D), lambda i: (i + _PREP_STEPS, 0)),
            pl.BlockSpec((_RB, 4), lambda i: (i, 0)),
            pl.BlockSpec((1, D), lambda i: (0, 0)),
            pl.BlockSpec((D, D), lambda i: (0, 0)),
            pl.BlockSpec((1, D), lambda i: (0, 0)),
            pl.BlockSpec((1, _RB), lambda i: (0, i)),
        ],
        out_specs=[
            pl.BlockSpec((_RB, D), lambda i: (i, 0)),
            pl.BlockSpec((_RB, D), lambda i: (i, 0)),
            pl.BlockSpec((_RB, D), lambda i: (i, 0)),
            pl.BlockSpec((G, D), lambda i: (0, 0)),
            pl.BlockSpec((G, D), lambda i: (0, 0)),
            pl.BlockSpec((G, D), lambda i: (0, 0)),
        ],
        out_shape=[
            jax.ShapeDtypeStruct((NPAD, D), jnp.float32),
            jax.ShapeDtypeStruct((NPAD, D), jnp.float32),
            jax.ShapeDtypeStruct((NPAD, D), jnp.float32),
            jax.ShapeDtypeStruct((G, D), jnp.float32),
            jax.ShapeDtypeStruct((G, D), jnp.float32),
            jax.ShapeDtypeStruct((G, D), jnp.float32),
        ],
    )(aggs, aggs, degt, fm, w, b2, batchpad)


def kernel(x, edge_index, batch, feat_mask, edge_mask, W, b):
    f32 = jnp.float32
    src = edge_index[0]
    dst = edge_index[1]
    npad_e = EPAD - E
    pad_iota = jax.lax.iota(jnp.int32, npad_e)
    src_pad = jnp.concatenate(
        [src, N + pad_iota % 128]).reshape(ER, 128)
    dst_pad = jnp.concatenate(
        [dst, pad_iota % N]).reshape(ER, 128)
    ones_pad = jnp.concatenate(
        [jnp.ones((E,), f32), jnp.zeros((npad_e,), f32)]).reshape(ER, 128)
    ew_pad = jnp.concatenate(
        [edge_mask.astype(f32), jnp.zeros((npad_e,), f32)]).reshape(ER, 128)
    vals = jnp.concatenate([ones_pad, ew_pad], axis=0)

    deg = _deg_kernel(src_pad, dst_pad, vals)
    degt = deg.reshape(4, NPAD).T

    xpad = jnp.pad(x, ((0, NPAD - N), (0, 0)))
    batchpad = jnp.pad(batch, (0, NPAD - N), constant_values=G).reshape(1, NPAD)
    fm = feat_mask.astype(f32)

    xs1, xs2, src2_pad, x1pad, m2, m1 = _prep_call(
        xpad, degt, fm, batchpad, src_pad, ew_pad)

    srcsel = jnp.concatenate(
        [src_pad, src2_pad], axis=0).reshape(2 * EPAD // _CH, _CH)
    zblk = jnp.zeros((_CH, D), f32)
    aggs = _agg_kernel(xs1, xs2, srcsel, dst_pad.reshape(EPAD // _CH, _CH),
                       zblk)

    b2 = b.astype(f32).reshape(1, D)
    zpad, z1pad, z2pad, g, g1, g2 = _post_call(
        aggs, degt, fm, W.astype(f32), b2, batchpad)

    return (zpad[:N], g, x1pad[:N], x, g1, g2, m1, m2)


# speedup vs baseline: 1.0288x; 1.0288x over previous
import functools

import jax
import jax.numpy as jnp
from jax import lax
from jax.experimental import pallas as pl
from jax.experimental.pallas import tpu as pltpu
from jax.experimental.pallas import tpu_sc as plsc

N = 10000
E = 320000
D = 128
G = 128

NPAD = 10240
EPAD = 327680
ER = EPAD // 128
TILES = 16
ER_T = ER // TILES
NR_T = NPAD // TILES

_MESH = plsc.VectorSubcoreMesh(core_axis_name="c", subcore_axis_name="s")


@functools.partial(
    pl.kernel,
    out_type=jax.ShapeDtypeStruct((4 * NPAD,), jnp.float32),
    mesh=_MESH,
    scratch_types=[
        pltpu.VMEM((ER_T, 128), jnp.int32),
        pltpu.VMEM((ER_T, 128), jnp.int32),
        pltpu.VMEM((ER_T, 128), jnp.float32),
        pltpu.VMEM((NR_T,), jnp.float32),
        pltpu.VMEM_SHARED((NPAD,), jnp.float32),
        pltpu.VMEM_SHARED((NPAD,), jnp.float32),
    ],
)
def _deg_kernel(src_hbm, dst_hbm, vals_hbm, deg_out,
                src_v, dst_v, vals_v, buf_v, hist_s, hist_d):
    c = lax.axis_index("c")
    s = lax.axis_index("s")

    def zero16(i, _):
        buf_v[pl.ds(i * 16, 16)] = jnp.zeros((16,), jnp.float32)
        return _

    lax.fori_loop(0, NR_T // 16, zero16, None)
    pltpu.sync_copy(buf_v, hist_s.at[pl.ds(s * NR_T, NR_T)])
    pltpu.sync_copy(buf_v, hist_d.at[pl.ds(s * NR_T, NR_T)])
    pltpu.sync_copy(src_hbm.at[pl.ds(s * ER_T, ER_T)], src_v)
    pltpu.sync_copy(dst_hbm.at[pl.ds(s * ER_T, ER_T)], dst_v)
    pltpu.sync_copy(vals_hbm.at[pl.ds((c * TILES + s) * ER_T, ER_T)], vals_v)
    plsc.subcore_barrier()

    def scat(j, _):
        pltpu.sync_copy(vals_v.at[j], hist_s.at[src_v.at[j]], add=True)
        pltpu.sync_copy(vals_v.at[j], hist_d.at[dst_v.at[j]], add=True)
        return _

    lax.fori_loop(0, ER_T, scat, None)
    plsc.subcore_barrier()
    pltpu.sync_copy(hist_s.at[pl.ds(s * NR_T, NR_T)], buf_v)
    pltpu.sync_copy(buf_v, deg_out.at[pl.ds((2 * c) * NPAD + s * NR_T, NR_T)])
    pltpu.sync_copy(hist_d.at[pl.ds(s * NR_T, NR_T)], buf_v)
    pltpu.sync_copy(buf_v, deg_out.at[pl.ds((2 * c + 1) * NPAD + s * NR_T, NR_T)])


_CH = 64
_NBUF = 4
_CH_T = EPAD // TILES // _CH
_GRP = _CH_T // _NBUF
_IDXROWS = 32


@functools.partial(
    pl.kernel,
    out_type=jax.ShapeDtypeStruct((2 * NPAD, D), jnp.float32),
    mesh=_MESH,
    scratch_types=[
        pltpu.VMEM((_IDXROWS, _CH), jnp.int32),
        pltpu.VMEM((_IDXROWS, _CH), jnp.int32),
        pltpu.VMEM((_CH, D), jnp.float32),
        pltpu.VMEM((_CH, D), jnp.float32),
        pltpu.VMEM((_CH, D), jnp.float32),
        pltpu.VMEM((_CH, D), jnp.float32),
        pltpu.VMEM_SHARED((NPAD, D), jnp.float32),
        pltpu.SemaphoreType.DMA,
        pltpu.SemaphoreType.DMA,
        pltpu.SemaphoreType.DMA,
        pltpu.SemaphoreType.DMA,
    ],
)
def _agg_kernel(xs1_hbm, xs2_hbm, src1_hbm, src2_hbm, dst_hbm, zblk_hbm,
                agg_out, src_v, dst_v, rows0, rows1, rows2, rows3, acc,
                sem0, sem1, sem2, sem3):
    c = lax.axis_index("c")
    s = lax.axis_index("s")
    rows = (rows0, rows1, rows2, rows3)
    sems = (sem0, sem1, sem2, sem3)

    pltpu.sync_copy(zblk_hbm, rows0)

    def zero_acc(k, _):
        pltpu.sync_copy(rows0, acc.at[pl.ds(s * NR_T + k * _CH, _CH)])
        return _

    lax.fori_loop(0, NR_T // _CH, zero_acc, None)
    plsc.subcore_barrier()

    def run(table_hbm, srcidx_hbm):
        def group(gk, _):
            @pl.when(gk % (_IDXROWS // _NBUF) == 0)
            def _():
                t = gk // (_IDXROWS // _NBUF)
                pltpu.sync_copy(
                    srcidx_hbm.at[pl.ds(s * _CH_T + t * _IDXROWS, _IDXROWS)],
                    src_v)
                pltpu.sync_copy(
                    dst_hbm.at[pl.ds(s * _CH_T + t * _IDXROWS, _IDXROWS)],
                    dst_v)

            base = (gk % (_IDXROWS // _NBUF)) * _NBUF
            gathers = []
            for b in range(_NBUF):
                @pl.when(gk > 0)
                def _(b=b):
                    pltpu.make_async_copy(zblk_hbm, rows[b], sems[b]).wait()

                gathers.append(pltpu.async_copy(
                    table_hbm.at[src_v.at[base + b]], rows[b], sems[b]))
            for b in range(_NBUF):
                gathers[b].wait()
                pltpu.async_copy(
                    rows[b], acc.at[dst_v.at[base + b]], sems[b], add=True)
            return _

        lax.fori_loop(0, _GRP, group, None)
        for b in range(_NBUF):
            pltpu.make_async_copy(zblk_hbm, rows[b], sems[b]).wait()

    @pl.when(c == 0)
    def _():
        run(xs1_hbm, src1_hbm)

    @pl.when(c == 1)
    def _():
        run(xs2_hbm, src2_hbm)

    plsc.subcore_barrier()

    def copy_out(k, _):
        base = s * NR_T + k * _CH
        pltpu.sync_copy(acc.at[pl.ds(base, _CH)], rows0)
        pltpu.sync_copy(rows0, agg_out.at[pl.ds(c * NPAD + base, _CH)])
        return _

    lax.fori_loop(0, NR_T // _CH, copy_out, None)


_PREP_STEPS = 10
_RB = NPAD // _PREP_STEPS
_EB = ER // _PREP_STEPS


def _prep_body(xpad_ref, degt_ref, fm_ref, batch_ref, src_ref, ew_ref,
               xs1_ref, xs2_ref, src2_ref, x1_ref, m2_ref, m1_ref):
    i = pl.program_id(0)
    xb = xpad_ref[...]
    inv1 = lax.rsqrt(jnp.maximum(degt_ref[:, 0:1], 1.0))
    inv2 = lax.rsqrt(jnp.maximum(degt_ref[:, 2:3], 1.0))
    xs1_ref[...] = xb * inv1
    xs2_ref[...] = xb * inv2
    x1_ref[...] = xb * fm_ref[...]
    zrow = N + lax.broadcasted_iota(jnp.int32, (_EB, 128), 1)
    src2_ref[...] = jnp.where(ew_ref[...] > 0.0, src_ref[...], zrow)
    gids = lax.broadcasted_iota(jnp.int32, (G, _RB), 0)
    oh = (gids == batch_ref[...]).astype(jnp.float32)

    @pl.when(i == 0)
    def _():
        m2_ref[...] = jnp.zeros((G, D), jnp.float32)

    m2_ref[...] += jnp.dot(oh, xb, preferred_element_type=jnp.float32)

    @pl.when(i == _PREP_STEPS - 1)
    def _():
        m1_ref[...] = m2_ref[...] * fm_ref[...]


def _prep_call(xpad, degt, fm, batchpad, src_pad, ew_pad):
    return pl.pallas_call(
        _prep_body,
        grid=(_PREP_STEPS,),
        in_specs=[
            pl.BlockSpec((_RB, D), lambda i: (i, 0)),
            pl.BlockSpec((_RB, 4), lambda i: (i, 0)),
            pl.BlockSpec((1, D), lambda i: (0, 0)),
            pl.BlockSpec((1, _RB), lambda i: (0, i)),
            pl.BlockSpec((_EB, 128), lambda i: (i, 0)),
            pl.BlockSpec((_EB, 128), lambda i: (i, 0)),
        ],
        out_specs=[
            pl.BlockSpec((_RB, D), lambda i: (i, 0)),
            pl.BlockSpec((_RB, D), lambda i: (i, 0)),
            pl.BlockSpec((_EB, 128), lambda i: (i, 0)),
            pl.BlockSpec((_RB, D), lambda i: (i, 0)),
            pl.BlockSpec((G, D), lambda i: (0, 0)),
            pl.BlockSpec((G, D), lambda i: (0, 0)),
        ],
        out_shape=[
            jax.ShapeDtypeStruct((NPAD, D), jnp.float32),
            jax.ShapeDtypeStruct((NPAD, D), jnp.float32),
            jax.ShapeDtypeStruct((ER, 128), jnp.int32),
            jax.ShapeDtypeStruct((NPAD, D), jnp.float32),
            jax.ShapeDtypeStruct((G, D), jnp.float32),
            jax.ShapeDtypeStruct((G, D), jnp.float32),
        ],
    )(xpad, degt, fm, batchpad, src_pad, ew_pad)


def _post_body(agg1_ref, agg2_ref, degt_ref, fm_ref, w_ref, b_ref, batch_ref,
               z_ref, z1_ref, z2_ref, g_ref, g1_ref, g2_ref):
    i = pl.program_id(0)
    invd1 = lax.rsqrt(jnp.maximum(degt_ref[:, 1:2], 1.0))
    invd2 = lax.rsqrt(jnp.maximum(degt_ref[:, 3:4], 1.0))
    a1 = agg1_ref[...] * invd1
    a2 = agg2_ref[...] * invd2
    w = w_ref[...]
    b = b_ref[...]
    z = jnp.maximum(jnp.dot(a1, w, preferred_element_type=jnp.float32) + b, 0.0)
    z1 = jnp.maximum(
        jnp.dot(a1 * fm_ref[...], w, preferred_element_type=jnp.float32) + b, 0.0)
    z2 = jnp.maximum(jnp.dot(a2, w, preferred_element_type=jnp.float32) + b, 0.0)
    z_ref[...] = z
    z1_ref[...] = z1
    z2_ref[...] = z2
    gids = lax.broadcasted_iota(jnp.int32, (G, _RB), 0)
    oh = (gids == batch_ref[...]).astype(jnp.float32)

    @pl.when(i == 0)
    def _():
        g_ref[...] = jnp.zeros((G, D), jnp.float32)
        g1_ref[...] = jnp.zeros((G, D), jnp.float32)
        g2_ref[...] = jnp.zeros((G, D), jnp.float32)

    g_ref[...] += jnp.dot(oh, z, preferred_element_type=jnp.float32)
    g1_ref[...] += jnp.dot(oh, z1, preferred_element_type=jnp.float32)
    g2_ref[...] += jnp.dot(oh, z2, preferred_element_type=jnp.float32)


def _post_call(aggs, degt, fm, w, b2, batchpad):
    return pl.pallas_call(
        _post_body,
        grid=(_PREP_STEPS,),
        in_specs=[
            pl.BlockSpec((_RB, D), lambda i: (i, 0)),
            pl.BlockSpec((_RB, D), lambda i: (i + _PREP_STEPS, 0)),
            pl.BlockSpec((_RB, 4), lambda i: (i, 0)),
            pl.BlockSpec((1, D), lambda i: (0, 0)),
            pl.BlockSpec((D, D), lambda i: (0, 0)),
            pl.BlockSpec((1, D), lambda i: (0, 0)),
            pl.BlockSpec((1, _RB), lambda i: (0, i)),
        ],
        out_specs=[
            pl.BlockSpec((_RB, D), lambda i: (i, 0)),
            pl.BlockSpec((_RB, D), lambda i: (i, 0)),
            pl.BlockSpec((_RB, D), lambda i: (i, 0)),
            pl.BlockSpec((G, D), lambda i: (0, 0)),
            pl.BlockSpec((G, D), lambda i: (0, 0)),
            pl.BlockSpec((G, D), lambda i: (0, 0)),
        ],
        out_shape=[
            jax.ShapeDtypeStruct((NPAD, D), jnp.float32),
            jax.ShapeDtypeStruct((NPAD, D), jnp.float32),
            jax.ShapeDtypeStruct((NPAD, D), jnp.float32),
            jax.ShapeDtypeStruct((G, D), jnp.float32),
            jax.ShapeDtypeStruct((G, D), jnp.float32),
            jax.ShapeDtypeStruct((G, D), jnp.float32),
        ],
    )(aggs, aggs, degt, fm, w, b2, batchpad)


def kernel(x, edge_index, batch, feat_mask, edge_mask, W, b):
    f32 = jnp.float32
    src = edge_index[0]
    dst = edge_index[1]
    npad_e = EPAD - E
    pad_iota = jax.lax.iota(jnp.int32, npad_e)
    src_pad = jnp.concatenate(
        [src, N + pad_iota % 128]).reshape(ER, 128)
    dst_pad = jnp.concatenate(
        [dst, pad_iota % N]).reshape(ER, 128)
    ones_pad = jnp.concatenate(
        [jnp.ones((E,), f32), jnp.zeros((npad_e,), f32)]).reshape(ER, 128)
    ew_pad = jnp.concatenate(
        [edge_mask.astype(f32), jnp.zeros((npad_e,), f32)]).reshape(ER, 128)
    vals = jnp.concatenate([ones_pad, ew_pad], axis=0)

    deg = _deg_kernel(src_pad, dst_pad, vals)
    degt = deg.reshape(4, NPAD).T

    xpad = jnp.pad(x, ((0, NPAD - N), (0, 0)))
    batchpad = jnp.pad(batch, (0, NPAD - N), constant_values=G).reshape(1, NPAD)
    fm = feat_mask.astype(f32)

    xs1, xs2, src2_pad, x1pad, m2, m1 = _prep_call(
        xpad, degt, fm, batchpad, src_pad, ew_pad)

    zblk = jnp.zeros((_CH, D), f32)
    aggs = _agg_kernel(xs1, xs2, src_pad.reshape(EPAD // _CH, _CH),
                       src2_pad.reshape(EPAD // _CH, _CH),
                       dst_pad.reshape(EPAD // _CH, _CH), zblk)

    b2 = b.astype(f32).reshape(1, D)
    zpad, z1pad, z2pad, g, g1, g2 = _post_call(
        aggs, degt, fm, W.astype(f32), b2, batchpad)

    return (zpad[:N], g, x1pad[:N], x, g1, g2, m1, m2)


# speedup vs baseline: 1.0914x; 1.0609x over previous
import functools

import jax
import jax.numpy as jnp
from jax import lax
from jax.experimental import pallas as pl
from jax.experimental.pallas import tpu as pltpu
from jax.experimental.pallas import tpu_sc as plsc

N = 10000
E = 320000
D = 128
G = 128

NPAD = 10240
EPAD = 327680
ER = EPAD // 128
TILES = 16
ER_T = ER // TILES
NR_T = NPAD // TILES

_MESH = plsc.VectorSubcoreMesh(core_axis_name="c", subcore_axis_name="s")


@functools.partial(
    pl.kernel,
    out_type=jax.ShapeDtypeStruct((4 * NPAD,), jnp.float32),
    mesh=_MESH,
    scratch_types=[
        pltpu.VMEM((ER_T, 128), jnp.int32),
        pltpu.VMEM((ER_T, 128), jnp.int32),
        pltpu.VMEM((ER_T, 128), jnp.float32),
        pltpu.VMEM((NR_T,), jnp.float32),
        pltpu.VMEM_SHARED((NPAD,), jnp.float32),
        pltpu.VMEM_SHARED((NPAD,), jnp.float32),
        pltpu.SemaphoreType.DMA,
        pltpu.SemaphoreType.DMA,
    ],
)
def _deg_kernel(src_hbm, dst_hbm, vals_hbm, deg_out,
                src_v, dst_v, vals_v, buf_v, hist_s, hist_d, sem_s, sem_d):
    c = lax.axis_index("c")
    s = lax.axis_index("s")

    def zero16(i, _):
        buf_v[pl.ds(i * 16, 16)] = jnp.zeros((16,), jnp.float32)
        return _

    lax.fori_loop(0, NR_T // 16, zero16, None)
    pltpu.sync_copy(buf_v, hist_s.at[pl.ds(s * NR_T, NR_T)])
    pltpu.sync_copy(buf_v, hist_d.at[pl.ds(s * NR_T, NR_T)])
    l1 = pltpu.async_copy(src_hbm.at[pl.ds(s * ER_T, ER_T)], src_v, sem_s)
    l2 = pltpu.async_copy(dst_hbm.at[pl.ds(s * ER_T, ER_T)], dst_v, sem_d)
    pltpu.sync_copy(vals_hbm.at[pl.ds((c * TILES + s) * ER_T, ER_T)], vals_v)
    l1.wait()
    l2.wait()
    plsc.subcore_barrier()

    def scat_grp(t, _):
        for u in range(16):
            j = t * 16 + u
            pltpu.async_copy(vals_v.at[j], hist_s.at[src_v.at[j]],
                             sem_s, add=True)
            pltpu.async_copy(vals_v.at[j], hist_d.at[dst_v.at[j]],
                             sem_d, add=True)
        pltpu.make_async_copy(
            src_hbm.at[pl.ds(0, 16)], src_v.at[pl.ds(0, 16)], sem_s).wait()
        pltpu.make_async_copy(
            dst_hbm.at[pl.ds(0, 16)], dst_v.at[pl.ds(0, 16)], sem_d).wait()
        return _

    lax.fori_loop(0, ER_T // 16, scat_grp, None)
    plsc.subcore_barrier()
    pltpu.sync_copy(hist_s.at[pl.ds(s * NR_T, NR_T)], buf_v)
    pltpu.sync_copy(buf_v, deg_out.at[pl.ds((2 * c) * NPAD + s * NR_T, NR_T)])
    pltpu.sync_copy(hist_d.at[pl.ds(s * NR_T, NR_T)], buf_v)
    pltpu.sync_copy(buf_v, deg_out.at[pl.ds((2 * c + 1) * NPAD + s * NR_T, NR_T)])


_CH = 64
_NBUF = 4
_CH_T = EPAD // TILES // _CH
_GRP = _CH_T // _NBUF
_IDXROWS = 32


@functools.partial(
    pl.kernel,
    out_type=jax.ShapeDtypeStruct((2 * NPAD, D), jnp.float32),
    mesh=_MESH,
    scratch_types=[
        pltpu.VMEM((_IDXROWS, _CH), jnp.int32),
        pltpu.VMEM((_IDXROWS, _CH), jnp.int32),
        pltpu.VMEM((_CH, D), jnp.float32),
        pltpu.VMEM((_CH, D), jnp.float32),
        pltpu.VMEM((_CH, D), jnp.float32),
        pltpu.VMEM((_CH, D), jnp.float32),
        pltpu.VMEM_SHARED((NPAD, D), jnp.float32),
        pltpu.SemaphoreType.DMA,
        pltpu.SemaphoreType.DMA,
        pltpu.SemaphoreType.DMA,
        pltpu.SemaphoreType.DMA,
    ],
)
def _agg_kernel(xs1_hbm, xs2_hbm, src1_hbm, src2_hbm, dst_hbm, zblk_hbm,
                agg_out, src_v, dst_v, rows0, rows1, rows2, rows3, acc,
                sem0, sem1, sem2, sem3):
    c = lax.axis_index("c")
    s = lax.axis_index("s")
    rows = (rows0, rows1, rows2, rows3)
    sems = (sem0, sem1, sem2, sem3)

    pltpu.sync_copy(zblk_hbm, rows0)

    def zero_acc(k, _):
        pltpu.sync_copy(rows0, acc.at[pl.ds(s * NR_T + k * _CH, _CH)])
        return _

    lax.fori_loop(0, NR_T // _CH, zero_acc, None)
    plsc.subcore_barrier()

    def run(table_hbm, srcidx_hbm):
        def group(gk, _):
            @pl.when(gk % (_IDXROWS // _NBUF) == 0)
            def _():
                t = gk // (_IDXROWS // _NBUF)
                pltpu.sync_copy(
                    srcidx_hbm.at[pl.ds(s * _CH_T + t * _IDXROWS, _IDXROWS)],
                    src_v)
                pltpu.sync_copy(
                    dst_hbm.at[pl.ds(s * _CH_T + t * _IDXROWS, _IDXROWS)],
                    dst_v)

            base = (gk % (_IDXROWS // _NBUF)) * _NBUF
            gathers = []
            for b in range(_NBUF):
                @pl.when(gk > 0)
                def _(b=b):
                    pltpu.make_async_copy(zblk_hbm, rows[b], sems[b]).wait()

                gathers.append(pltpu.async_copy(
                    table_hbm.at[src_v.at[base + b]], rows[b], sems[b]))
            for b in range(_NBUF):
                gathers[b].wait()
                pltpu.async_copy(
                    rows[b], acc.at[dst_v.at[base + b]], sems[b], add=True)
            return _

        lax.fori_loop(0, _GRP, group, None)
        for b in range(_NBUF):
            pltpu.make_async_copy(zblk_hbm, rows[b], sems[b]).wait()

    @pl.when(c == 0)
    def _():
        run(xs1_hbm, src1_hbm)

    @pl.when(c == 1)
    def _():
        run(xs2_hbm, src2_hbm)

    plsc.subcore_barrier()

    def copy_out(k, _):
        base = s * NR_T + k * _CH
        pltpu.sync_copy(acc.at[pl.ds(base, _CH)], rows0)
        pltpu.sync_copy(rows0, agg_out.at[pl.ds(c * NPAD + base, _CH)])
        return _

    lax.fori_loop(0, NR_T // _CH, copy_out, None)


_PREP_STEPS = 10
_RB = NPAD // _PREP_STEPS
_EB = ER // _PREP_STEPS


def _prep_body(xpad_ref, degt_ref, fm_ref, batch_ref, src_ref, ew_ref,
               xs1_ref, xs2_ref, src2_ref, x1_ref, m2_ref, m1_ref):
    i = pl.program_id(0)
    xb = xpad_ref[...]
    inv1 = lax.rsqrt(jnp.maximum(degt_ref[:, 0:1], 1.0))
    inv2 = lax.rsqrt(jnp.maximum(degt_ref[:, 2:3], 1.0))
    xs1_ref[...] = xb * inv1
    xs2_ref[...] = xb * inv2
    x1_ref[...] = xb * fm_ref[...]
    zrow = N + lax.broadcasted_iota(jnp.int32, (_EB, 128), 1)
    src2_ref[...] = jnp.where(ew_ref[...] > 0.0, src_ref[...], zrow)
    gids = lax.broadcasted_iota(jnp.int32, (G, _RB), 0)
    oh = (gids == batch_ref[...]).astype(jnp.float32)

    @pl.when(i == 0)
    def _():
        m2_ref[...] = jnp.zeros((G, D), jnp.float32)

    m2_ref[...] += jnp.dot(oh, xb, preferred_element_type=jnp.float32)

    @pl.when(i == _PREP_STEPS - 1)
    def _():
        m1_ref[...] = m2_ref[...] * fm_ref[...]


def _prep_call(xpad, degt, fm, batchpad, src_pad, ew_pad):
    return pl.pallas_call(
        _prep_body,
        grid=(_PREP_STEPS,),
        in_specs=[
            pl.BlockSpec((_RB, D), lambda i: (i, 0)),
            pl.BlockSpec((_RB, 4), lambda i: (i, 0)),
            pl.BlockSpec((1, D), lambda i: (0, 0)),
            pl.BlockSpec((1, _RB), lambda i: (0, i)),
            pl.BlockSpec((_EB, 128), lambda i: (i, 0)),
            pl.BlockSpec((_EB, 128), lambda i: (i, 0)),
        ],
        out_specs=[
            pl.BlockSpec((_RB, D), lambda i: (i, 0)),
            pl.BlockSpec((_RB, D), lambda i: (i, 0)),
            pl.BlockSpec((_EB, 128), lambda i: (i, 0)),
            pl.BlockSpec((_RB, D), lambda i: (i, 0)),
            pl.BlockSpec((G, D), lambda i: (0, 0)),
            pl.BlockSpec((G, D), lambda i: (0, 0)),
        ],
        out_shape=[
            jax.ShapeDtypeStruct((NPAD, D), jnp.float32),
            jax.ShapeDtypeStruct((NPAD, D), jnp.float32),
            jax.ShapeDtypeStruct((ER, 128), jnp.int32),
            jax.ShapeDtypeStruct((NPAD, D), jnp.float32),
            jax.ShapeDtypeStruct((G, D), jnp.float32),
            jax.ShapeDtypeStruct((G, D), jnp.float32),
        ],
    )(xpad, degt, fm, batchpad, src_pad, ew_pad)


def _post_body(agg1_ref, agg2_ref, degt_ref, fm_ref, w_ref, b_ref, batch_ref,
               z_ref, z1_ref, z2_ref, g_ref, g1_ref, g2_ref):
    i = pl.program_id(0)
    invd1 = lax.rsqrt(jnp.maximum(degt_ref[:, 1:2], 1.0))
    invd2 = lax.rsqrt(jnp.maximum(degt_ref[:, 3:4], 1.0))
    a1 = agg1_ref[...] * invd1
    a2 = agg2_ref[...] * invd2
    w = w_ref[...]
    b = b_ref[...]
    z = jnp.maximum(jnp.dot(a1, w, preferred_element_type=jnp.float32) + b, 0.0)
    z1 = jnp.maximum(
        jnp.dot(a1 * fm_ref[...], w, preferred_element_type=jnp.float32) + b, 0.0)
    z2 = jnp.maximum(jnp.dot(a2, w, preferred_element_type=jnp.float32) + b, 0.0)
    z_ref[...] = z
    z1_ref[...] = z1
    z2_ref[...] = z2
    gids = lax.broadcasted_iota(jnp.int32, (G, _RB), 0)
    oh = (gids == batch_ref[...]).astype(jnp.float32)

    @pl.when(i == 0)
    def _():
        g_ref[...] = jnp.zeros((G, D), jnp.float32)
        g1_ref[...] = jnp.zeros((G, D), jnp.float32)
        g2_ref[...] = jnp.zeros((G, D), jnp.float32)

    g_ref[...] += jnp.dot(oh, z, preferred_element_type=jnp.float32)
    g1_ref[...] += jnp.dot(oh, z1, preferred_element_type=jnp.float32)
    g2_ref[...] += jnp.dot(oh, z2, preferred_element_type=jnp.float32)


def _post_call(aggs, degt, fm, w, b2, batchpad):
    return pl.pallas_call(
        _post_body,
        grid=(_PREP_STEPS,),
        in_specs=[
            pl.BlockSpec((_RB, D), lambda i: (i, 0)),
            pl.BlockSpec((_RB, D), lambda i: (i + _PREP_STEPS, 0)),
            pl.BlockSpec((_RB, 4), lambda i: (i, 0)),
            pl.BlockSpec((1, D), lambda i: (0, 0)),
            pl.BlockSpec((D, D), lambda i: (0, 0)),
            pl.BlockSpec((1, D), lambda i: (0, 0)),
            pl.BlockSpec((1, _RB), lambda i: (0, i)),
        ],
        out_specs=[
            pl.BlockSpec((_RB, D), lambda i: (i, 0)),
            pl.BlockSpec((_RB, D), lambda i: (i, 0)),
            pl.BlockSpec((_RB, D), lambda i: (i, 0)),
            pl.BlockSpec((G, D), lambda i: (0, 0)),
            pl.BlockSpec((G, D), lambda i: (0, 0)),
            pl.BlockSpec((G, D), lambda i: (0, 0)),
        ],
        out_shape=[
            jax.ShapeDtypeStruct((NPAD, D), jnp.float32),
            jax.ShapeDtypeStruct((NPAD, D), jnp.float32),
            jax.ShapeDtypeStruct((NPAD, D), jnp.float32),
            jax.ShapeDtypeStruct((G, D), jnp.float32),
            jax.ShapeDtypeStruct((G, D), jnp.float32),
            jax.ShapeDtypeStruct((G, D), jnp.float32),
        ],
    )(aggs, aggs, degt, fm, w, b2, batchpad)


def kernel(x, edge_index, batch, feat_mask, edge_mask, W, b):
    f32 = jnp.float32
    src = edge_index[0]
    dst = edge_index[1]
    npad_e = EPAD - E
    pad_iota = jax.lax.iota(jnp.int32, npad_e)
    src_pad = jnp.concatenate(
        [src, N + pad_iota % 128]).reshape(ER, 128)
    dst_pad = jnp.concatenate(
        [dst, pad_iota % N]).reshape(ER, 128)
    ones_pad = jnp.concatenate(
        [jnp.ones((E,), f32), jnp.zeros((npad_e,), f32)]).reshape(ER, 128)
    ew_pad = jnp.concatenate(
        [edge_mask.astype(f32), jnp.zeros((npad_e,), f32)]).reshape(ER, 128)
    vals = jnp.concatenate([ones_pad, ew_pad], axis=0)

    deg = _deg_kernel(src_pad, dst_pad, vals)
    degt = deg.reshape(4, NPAD).T

    xpad = jnp.pad(x, ((0, NPAD - N), (0, 0)))
    batchpad = jnp.pad(batch, (0, NPAD - N), constant_values=G).reshape(1, NPAD)
    fm = feat_mask.astype(f32)

    xs1, xs2, src2_pad, x1pad, m2, m1 = _prep_call(
        xpad, degt, fm, batchpad, src_pad, ew_pad)

    zblk = jnp.zeros((_CH, D), f32)
    aggs = _agg_kernel(xs1, xs2, src_pad.reshape(EPAD // _CH, _CH),
                       src2_pad.reshape(EPAD // _CH, _CH),
                       dst_pad.reshape(EPAD // _CH, _CH), zblk)

    b2 = b.astype(f32).reshape(1, D)
    zpad, z1pad, z2pad, g, g1, g2 = _post_call(
        aggs, degt, fm, W.astype(f32), b2, batchpad)

    return (zpad[:N], g, x1pad[:N], x, g1, g2, m1, m2)


# speedup vs baseline: 1.1103x; 1.0173x over previous
import functools

import jax
import jax.numpy as jnp
from jax import lax
from jax.experimental import pallas as pl
from jax.experimental.pallas import tpu as pltpu
from jax.experimental.pallas import tpu_sc as plsc

N = 10000
E = 320000
D = 128
G = 128

NPAD = 10240
EPAD = 327680
ER = EPAD // 128
TILES = 16
ER_T = ER // TILES
NR_T = NPAD // TILES

_MESH = plsc.VectorSubcoreMesh(core_axis_name="c", subcore_axis_name="s")


@functools.partial(
    pl.kernel,
    out_type=jax.ShapeDtypeStruct((4 * NPAD,), jnp.float32),
    mesh=_MESH,
    scratch_types=[
        pltpu.VMEM((ER_T, 128), jnp.int32),
        pltpu.VMEM((ER_T, 128), jnp.int32),
        pltpu.VMEM((ER_T, 128), jnp.float32),
        pltpu.VMEM((NR_T,), jnp.float32),
        pltpu.VMEM_SHARED((NPAD,), jnp.float32),
        pltpu.VMEM_SHARED((NPAD,), jnp.float32),
        pltpu.SemaphoreType.DMA,
        pltpu.SemaphoreType.DMA,
    ],
)
def _deg_kernel(src_hbm, dst_hbm, vals_hbm, deg_out,
                src_v, dst_v, vals_v, buf_v, hist_s, hist_d, sem_s, sem_d):
    c = lax.axis_index("c")
    s = lax.axis_index("s")

    def zero16(i, _):
        buf_v[pl.ds(i * 16, 16)] = jnp.zeros((16,), jnp.float32)
        return _

    lax.fori_loop(0, NR_T // 16, zero16, None)
    pltpu.sync_copy(buf_v, hist_s.at[pl.ds(s * NR_T, NR_T)])
    pltpu.sync_copy(buf_v, hist_d.at[pl.ds(s * NR_T, NR_T)])
    l1 = pltpu.async_copy(src_hbm.at[pl.ds(s * ER_T, ER_T)], src_v, sem_s)
    l2 = pltpu.async_copy(dst_hbm.at[pl.ds(s * ER_T, ER_T)], dst_v, sem_d)
    pltpu.sync_copy(vals_hbm.at[pl.ds((c * TILES + s) * ER_T, ER_T)], vals_v)
    l1.wait()
    l2.wait()
    plsc.subcore_barrier()

    def scat_grp(t, _):
        for u in range(16):
            j = t * 16 + u
            pltpu.async_copy(vals_v.at[j], hist_s.at[src_v.at[j]],
                             sem_s, add=True)
            pltpu.async_copy(vals_v.at[j], hist_d.at[dst_v.at[j]],
                             sem_d, add=True)
        pltpu.make_async_copy(
            src_hbm.at[pl.ds(0, 16)], src_v.at[pl.ds(0, 16)], sem_s).wait()
        pltpu.make_async_copy(
            dst_hbm.at[pl.ds(0, 16)], dst_v.at[pl.ds(0, 16)], sem_d).wait()
        return _

    lax.fori_loop(0, ER_T // 16, scat_grp, None)
    plsc.subcore_barrier()
    pltpu.sync_copy(hist_s.at[pl.ds(s * NR_T, NR_T)], buf_v)
    pltpu.sync_copy(buf_v, deg_out.at[pl.ds((2 * c) * NPAD + s * NR_T, NR_T)])
    pltpu.sync_copy(hist_d.at[pl.ds(s * NR_T, NR_T)], buf_v)
    pltpu.sync_copy(buf_v, deg_out.at[pl.ds((2 * c + 1) * NPAD + s * NR_T, NR_T)])


_CH = 64
_NBUF = 4
_CH_T = EPAD // TILES // _CH
_GRP = _CH_T // _NBUF
_IDXROWS = 32


@functools.partial(
    pl.kernel,
    out_type=jax.ShapeDtypeStruct((2 * NPAD, D), jnp.float32),
    mesh=_MESH,
    scratch_types=[
        pltpu.VMEM((_IDXROWS, _CH), jnp.int32),
        pltpu.VMEM((_IDXROWS, _CH), jnp.int32),
        pltpu.VMEM((_CH, D), jnp.float32),
        pltpu.VMEM((_CH, D), jnp.float32),
        pltpu.VMEM((_CH, D), jnp.float32),
        pltpu.VMEM((_CH, D), jnp.float32),
        pltpu.VMEM_SHARED((NPAD, D), jnp.float32),
        pltpu.SemaphoreType.DMA,
        pltpu.SemaphoreType.DMA,
        pltpu.SemaphoreType.DMA,
        pltpu.SemaphoreType.DMA,
        pltpu.SemaphoreType.DMA,
    ],
)
def _agg_kernel(xs1_hbm, xs2_hbm, src1_hbm, src2_hbm, dst_hbm, zblk_hbm,
                agg_out, src_v, dst_v, rows0, rows1, rows2, rows3, acc,
                sem0, sem1, sem2, sem3, sem_i):
    c = lax.axis_index("c")
    s = lax.axis_index("s")
    rows = (rows0, rows1, rows2, rows3)
    sems = (sem0, sem1, sem2, sem3)

    pltpu.sync_copy(zblk_hbm, rows0)

    def zero_acc(k, _):
        pltpu.sync_copy(rows0, acc.at[pl.ds(s * NR_T + k * _CH, _CH)])
        return _

    lax.fori_loop(0, NR_T // _CH, zero_acc, None)
    plsc.subcore_barrier()

    def run(table_hbm, srcidx_hbm):
        def group(gk, _):
            @pl.when(gk % (_IDXROWS // _NBUF) == 0)
            def _():
                t = gk // (_IDXROWS // _NBUF)
                l1 = pltpu.async_copy(
                    srcidx_hbm.at[pl.ds(s * _CH_T + t * _IDXROWS, _IDXROWS)],
                    src_v, sem_i)
                l2 = pltpu.async_copy(
                    dst_hbm.at[pl.ds(s * _CH_T + t * _IDXROWS, _IDXROWS)],
                    dst_v, sem_i)
                l1.wait()
                l2.wait()

            base = (gk % (_IDXROWS // _NBUF)) * _NBUF
            gathers = []
            for b in range(_NBUF):
                @pl.when(gk > 0)
                def _(b=b):
                    pltpu.make_async_copy(zblk_hbm, rows[b], sems[b]).wait()

                gathers.append(pltpu.async_copy(
                    table_hbm.at[src_v.at[base + b]], rows[b], sems[b]))
            for b in range(_NBUF):
                gathers[b].wait()
                pltpu.async_copy(
                    rows[b], acc.at[dst_v.at[base + b]], sems[b], add=True)
            return _

        lax.fori_loop(0, _GRP, group, None)
        for b in range(_NBUF):
            pltpu.make_async_copy(zblk_hbm, rows[b], sems[b]).wait()

    @pl.when(c == 0)
    def _():
        run(xs1_hbm, src1_hbm)

    @pl.when(c == 1)
    def _():
        run(xs2_hbm, src2_hbm)

    plsc.subcore_barrier()

    def copy_out(k, _):
        base = s * NR_T + k * _CH
        pltpu.sync_copy(acc.at[pl.ds(base, _CH)], rows0)
        pltpu.sync_copy(rows0, agg_out.at[pl.ds(c * NPAD + base, _CH)])
        return _

    lax.fori_loop(0, NR_T // _CH, copy_out, None)


_PREP_STEPS = 10
_RB = NPAD // _PREP_STEPS
_EB = ER // _PREP_STEPS


def _prep_body(xpad_ref, degt_ref, fm_ref, batch_ref, src_ref, ew_ref,
               xs1_ref, xs2_ref, src2_ref, x1_ref, m2_ref, m1_ref):
    i = pl.program_id(0)
    xb = xpad_ref[...]
    inv1 = lax.rsqrt(jnp.maximum(degt_ref[:, 0:1], 1.0))
    inv2 = lax.rsqrt(jnp.maximum(degt_ref[:, 2:3], 1.0))
    xs1_ref[...] = xb * inv1
    xs2_ref[...] = xb * inv2
    x1_ref[...] = xb * fm_ref[...]
    zrow = N + lax.broadcasted_iota(jnp.int32, (_EB, 128), 1)
    src2_ref[...] = jnp.where(ew_ref[...] > 0.0, src_ref[...], zrow)
    gids = lax.broadcasted_iota(jnp.int32, (G, _RB), 0)
    oh = (gids == batch_ref[...]).astype(jnp.float32)

    @pl.when(i == 0)
    def _():
        m2_ref[...] = jnp.zeros((G, D), jnp.float32)

    m2_ref[...] += jnp.dot(oh, xb, preferred_element_type=jnp.float32)

    @pl.when(i == _PREP_STEPS - 1)
    def _():
        m1_ref[...] = m2_ref[...] * fm_ref[...]


def _prep_call(xpad, degt, fm, batchpad, src_pad, ew_pad):
    return pl.pallas_call(
        _prep_body,
        grid=(_PREP_STEPS,),
        in_specs=[
            pl.BlockSpec((_RB, D), lambda i: (i, 0)),
            pl.BlockSpec((_RB, 4), lambda i: (i, 0)),
            pl.BlockSpec((1, D), lambda i: (0, 0)),
            pl.BlockSpec((1, _RB), lambda i: (0, i)),
            pl.BlockSpec((_EB, 128), lambda i: (i, 0)),
            pl.BlockSpec((_EB, 128), lambda i: (i, 0)),
        ],
        out_specs=[
            pl.BlockSpec((_RB, D), lambda i: (i, 0)),
            pl.BlockSpec((_RB, D), lambda i: (i, 0)),
            pl.BlockSpec((_EB, 128), lambda i: (i, 0)),
            pl.BlockSpec((_RB, D), lambda i: (i, 0)),
            pl.BlockSpec((G, D), lambda i: (0, 0)),
            pl.BlockSpec((G, D), lambda i: (0, 0)),
        ],
        out_shape=[
            jax.ShapeDtypeStruct((NPAD, D), jnp.float32),
            jax.ShapeDtypeStruct((NPAD, D), jnp.float32),
            jax.ShapeDtypeStruct((ER, 128), jnp.int32),
            jax.ShapeDtypeStruct((NPAD, D), jnp.float32),
            jax.ShapeDtypeStruct((G, D), jnp.float32),
            jax.ShapeDtypeStruct((G, D), jnp.float32),
        ],
    )(xpad, degt, fm, batchpad, src_pad, ew_pad)


def _post_body(agg1_ref, agg2_ref, degt_ref, fm_ref, w_ref, b_ref, batch_ref,
               z_ref, z1_ref, z2_ref, g_ref, g1_ref, g2_ref):
    i = pl.program_id(0)
    invd1 = lax.rsqrt(jnp.maximum(degt_ref[:, 1:2], 1.0))
    invd2 = lax.rsqrt(jnp.maximum(degt_ref[:, 3:4], 1.0))
    a1 = agg1_ref[...] * invd1
    a2 = agg2_ref[...] * invd2
    w = w_ref[...]
    b = b_ref[...]
    z = jnp.maximum(jnp.dot(a1, w, preferred_element_type=jnp.float32) + b, 0.0)
    z1 = jnp.maximum(
        jnp.dot(a1 * fm_ref[...], w, preferred_element_type=jnp.float32) + b, 0.0)
    z2 = jnp.maximum(jnp.dot(a2, w, preferred_element_type=jnp.float32) + b, 0.0)
    z_ref[...] = z
    z1_ref[...] = z1
    z2_ref[...] = z2
    gids = lax.broadcasted_iota(jnp.int32, (G, _RB), 0)
    oh = (gids == batch_ref[...]).astype(jnp.float32)

    @pl.when(i == 0)
    def _():
        g_ref[...] = jnp.zeros((G, D), jnp.float32)
        g1_ref[...] = jnp.zeros((G, D), jnp.float32)
        g2_ref[...] = jnp.zeros((G, D), jnp.float32)

    g_ref[...] += jnp.dot(oh, z, preferred_element_type=jnp.float32)
    g1_ref[...] += jnp.dot(oh, z1, preferred_element_type=jnp.float32)
    g2_ref[...] += jnp.dot(oh, z2, preferred_element_type=jnp.float32)


def _post_call(aggs, degt, fm, w, b2, batchpad):
    return pl.pallas_call(
        _post_body,
        grid=(_PREP_STEPS,),
        in_specs=[
            pl.BlockSpec((_RB, D), lambda i: (i, 0)),
            pl.BlockSpec((_RB, D), lambda i: (i + _PREP_STEPS, 0)),
            pl.BlockSpec((_RB, 4), lambda i: (i, 0)),
            pl.BlockSpec((1, D), lambda i: (0, 0)),
            pl.BlockSpec((D, D), lambda i: (0, 0)),
            pl.BlockSpec((1, D), lambda i: (0, 0)),
            pl.BlockSpec((1, _RB), lambda i: (0, i)),
        ],
        out_specs=[
            pl.BlockSpec((_RB, D), lambda i: (i, 0)),
            pl.BlockSpec((_RB, D), lambda i: (i, 0)),
            pl.BlockSpec((_RB, D), lambda i: (i, 0)),
            pl.BlockSpec((G, D), lambda i: (0, 0)),
            pl.BlockSpec((G, D), lambda i: (0, 0)),
            pl.BlockSpec((G, D), lambda i: (0, 0)),
        ],
        out_shape=[
            jax.ShapeDtypeStruct((NPAD, D), jnp.float32),
            jax.ShapeDtypeStruct((NPAD, D), jnp.float32),
            jax.ShapeDtypeStruct((NPAD, D), jnp.float32),
            jax.ShapeDtypeStruct((G, D), jnp.float32),
            jax.ShapeDtypeStruct((G, D), jnp.float32),
            jax.ShapeDtypeStruct((G, D), jnp.float32),
        ],
    )(aggs, aggs, degt, fm, w, b2, batchpad)


def kernel(x, edge_index, batch, feat_mask, edge_mask, W, b):
    f32 = jnp.float32
    src = edge_index[0]
    dst = edge_index[1]
    npad_e = EPAD - E
    pad_iota = jax.lax.iota(jnp.int32, npad_e)
    src_pad = jnp.concatenate(
        [src, N + pad_iota % 128]).reshape(ER, 128)
    dst_pad = jnp.concatenate(
        [dst, pad_iota % N]).reshape(ER, 128)
    ones_pad = jnp.concatenate(
        [jnp.ones((E,), f32), jnp.zeros((npad_e,), f32)]).reshape(ER, 128)
    ew_pad = jnp.concatenate(
        [edge_mask.astype(f32), jnp.zeros((npad_e,), f32)]).reshape(ER, 128)
    vals = jnp.concatenate([ones_pad, ew_pad], axis=0)

    deg = _deg_kernel(src_pad, dst_pad, vals)
    degt = deg.reshape(4, NPAD).T

    xpad = jnp.pad(x, ((0, NPAD - N), (0, 0)))
    batchpad = jnp.pad(batch, (0, NPAD - N), constant_values=G).reshape(1, NPAD)
    fm = feat_mask.astype(f32)

    xs1, xs2, src2_pad, x1pad, m2, m1 = _prep_call(
        xpad, degt, fm, batchpad, src_pad, ew_pad)

    zblk = jnp.zeros((_CH, D), f32)
    aggs = _agg_kernel(xs1, xs2, src_pad.reshape(EPAD // _CH, _CH),
                       src2_pad.reshape(EPAD // _CH, _CH),
                       dst_pad.reshape(EPAD // _CH, _CH), zblk)

    b2 = b.astype(f32).reshape(1, D)
    zpad, z1pad, z2pad, g, g1, g2 = _post_call(
        aggs, degt, fm, W.astype(f32), b2, batchpad)

    return (zpad[:N], g, x1pad[:N], x, g1, g2, m1, m2)
